# Initial kernel scaffold; baseline (speedup 1.0000x reference)
import jax
import jax.numpy as jnp
from jax.experimental import pallas as pl


def kernel(x, table):
    raise NotImplementedError("write your pallas kernel here")


# speedup vs baseline: 3.8377x; 3.8377x over previous
import functools

import jax
import jax.numpy as jnp
from jax import lax
from jax.experimental import pallas as pl
from jax.experimental.pallas import tpu as pltpu
from jax.experimental.pallas import tpu_sc as plsc

EMBEDDING_LENGTH = 128
VOCAB = 256

NUM_CORES = 2
NUM_SUBCORES = 16
NW = NUM_CORES * NUM_SUBCORES

CHUNK = 128


def _make_kernel(n_rows: int):
    assert n_rows % (NW * CHUNK) == 0
    chunks_per_w = n_rows // (NW * CHUNK)
    mesh = plsc.VectorSubcoreMesh(
        core_axis_name="c", subcore_axis_name="s",
        num_cores=NUM_CORES, num_subcores=NUM_SUBCORES)

    @functools.partial(
        pl.kernel,
        out_type=jax.ShapeDtypeStruct((n_rows, EMBEDDING_LENGTH), jnp.float32),
        mesh=mesh,
        scratch_types=[
            pltpu.VMEM((CHUNK,), jnp.int32),
            pltpu.VMEM((CHUNK, EMBEDDING_LENGTH), jnp.float32),
            pltpu.SemaphoreType.DMA,
        ],
    )
    def gather_kernel(table_hbm, idx_hbm, out_hbm, idx_v, rows_v, sem):
        wid = lax.axis_index("s") * NUM_CORES + lax.axis_index("c")
        base_chunk = wid * chunks_per_w

        def body(j, carry):
            chunk = base_chunk + j
            pltpu.sync_copy(idx_hbm.at[chunk], idx_v)
            pltpu.async_copy(table_hbm.at[idx_v], rows_v, sem).wait()
            pltpu.sync_copy(rows_v, out_hbm.at[pl.ds(chunk * CHUNK, CHUNK)])
            return carry

        lax.fori_loop(0, chunks_per_w, body, 0)

    return gather_kernel


def kernel(x, table):
    orig_shape = x.shape
    n_rows = x.size
    idx = x.reshape(n_rows // CHUNK, CHUNK).astype(jnp.int32)
    out = _make_kernel(n_rows)(table, idx)
    return out.reshape(*orig_shape, EMBEDDING_LENGTH)


# speedup vs baseline: 15.4658x; 4.0300x over previous
import functools

import jax
import jax.numpy as jnp
from jax import lax
from jax.experimental import pallas as pl
from jax.experimental.pallas import tpu as pltpu
from jax.experimental.pallas import tpu_sc as plsc

EMBEDDING_LENGTH = 128
VOCAB = 256

NUM_CORES = 2
NUM_SUBCORES = 16
NW = NUM_CORES * NUM_SUBCORES

CHUNK = 128
NBUF = 4


def _make_kernel(n_rows: int):
    assert n_rows % (NW * CHUNK) == 0
    chunks_per_w = n_rows // (NW * CHUNK)
    assert chunks_per_w % NBUF == 0
    mesh = plsc.VectorSubcoreMesh(
        core_axis_name="c", subcore_axis_name="s",
        num_cores=NUM_CORES, num_subcores=NUM_SUBCORES)

    @functools.partial(
        pl.kernel,
        out_type=jax.ShapeDtypeStruct((n_rows, EMBEDDING_LENGTH), jnp.float32),
        mesh=mesh,
        scratch_types=[
            pltpu.VMEM((chunks_per_w, CHUNK), jnp.int32),
            pltpu.VMEM_SHARED((VOCAB, EMBEDDING_LENGTH), jnp.float32),
        ] + [pltpu.VMEM((CHUNK, EMBEDDING_LENGTH), jnp.float32)] * NBUF
          + [pltpu.SemaphoreType.DMA] * (2 * NBUF),
    )
    def gather_kernel(table_hbm, idx_hbm, out_hbm, idx_v, table_v, *rest):
        bufs = rest[:NBUF]
        gsems = rest[NBUF:2 * NBUF]
        wsems = rest[2 * NBUF:]
        sid = lax.axis_index("s")
        wid = sid * NUM_CORES + lax.axis_index("c")
        base = wid * chunks_per_w

        pltpu.sync_copy(idx_hbm.at[pl.ds(base, chunks_per_w)], idx_v)

        @pl.when(sid == 0)
        def _stage_table():
            pltpu.sync_copy(table_hbm, table_v)

        plsc.subcore_barrier()

        def out_slice(j):
            return out_hbm.at[pl.ds((base + j) * CHUNK, CHUNK)]

        def fire_pair(g, pair):
            return [
                pltpu.async_copy(
                    table_v.at[idx_v.at[2 * g + i]],
                    bufs[2 * pair + i], gsems[2 * pair + i])
                for i in range(2)
            ]

        def finish_pair(g, pair, handles):
            for i in range(2):
                handles[i].wait()
                pltpu.async_copy(
                    bufs[2 * pair + i], out_slice(2 * g + i),
                    wsems[2 * pair + i])

        def drain_pair(g, pair):
            for i in range(2):
                pltpu.make_async_copy(
                    bufs[2 * pair + i], out_slice(2 * g + i),
                    wsems[2 * pair + i]).wait()

        finish_pair(0, 0, fire_pair(0, 0))
        finish_pair(1, 1, fire_pair(1, 1))

        def body(k, carry):
            g0 = 2 * k
            drain_pair(g0 - 2, 0)
            finish_pair(g0, 0, fire_pair(g0, 0))
            drain_pair(g0 - 1, 1)
            finish_pair(g0 + 1, 1, fire_pair(g0 + 1, 1))
            return carry

        n_pairs = chunks_per_w // 2
        lax.fori_loop(1, n_pairs // 2, body, 0)

        drain_pair(n_pairs - 2, 0)
        drain_pair(n_pairs - 1, 1)

    return gather_kernel


def kernel(x, table):
    orig_shape = x.shape
    n_rows = x.size
    idx = x.reshape(n_rows // CHUNK, CHUNK).astype(jnp.int32)
    out = _make_kernel(n_rows)(table, idx)
    return out.reshape(*orig_shape, EMBEDDING_LENGTH)


# speedup vs baseline: 15.7742x; 1.0199x over previous
import functools

import jax
import jax.numpy as jnp
from jax import lax
from jax.experimental import pallas as pl
from jax.experimental.pallas import tpu as pltpu
from jax.experimental.pallas import tpu_sc as plsc

EMBEDDING_LENGTH = 128
VOCAB = 256

NUM_CORES = 2
NUM_SUBCORES = 16
NW = NUM_CORES * NUM_SUBCORES

CHUNK = 128
NBUF = 4


def _make_kernel(n_rows: int):
    assert n_rows % (NW * CHUNK) == 0
    chunks_per_w = n_rows // (NW * CHUNK)
    assert chunks_per_w % NBUF == 0
    mesh = plsc.VectorSubcoreMesh(
        core_axis_name="c", subcore_axis_name="s",
        num_cores=NUM_CORES, num_subcores=NUM_SUBCORES)

    @functools.partial(
        pl.kernel,
        out_type=jax.ShapeDtypeStruct((n_rows, EMBEDDING_LENGTH), jnp.float32),
        mesh=mesh,
        scratch_types=[
            pltpu.VMEM((chunks_per_w, CHUNK), jnp.int32),
            pltpu.VMEM_SHARED((VOCAB, EMBEDDING_LENGTH), jnp.float32),
        ] + [pltpu.VMEM((CHUNK, EMBEDDING_LENGTH), jnp.float32)] * NBUF
          + [pltpu.SemaphoreType.DMA] * (2 * NBUF),
    )
    def gather_kernel(table_hbm, idx_hbm, out_hbm, idx_v, table_v, *rest):
        bufs = rest[:NBUF]
        gsems = rest[NBUF:2 * NBUF]
        wsems = rest[2 * NBUF:]
        sid = lax.axis_index("s")
        wid = sid * NUM_CORES + lax.axis_index("c")
        base = wid * chunks_per_w

        pltpu.sync_copy(idx_hbm.at[pl.ds(base, chunks_per_w)], idx_v)

        @pl.when(sid == 0)
        def _stage_table():
            pltpu.sync_copy(table_hbm, table_v)

        plsc.subcore_barrier()

        def out_slice(j):
            return out_hbm.at[pl.ds((base + j) * CHUNK, CHUNK)]

        def fire_pair(g, pair):
            return [
                pltpu.async_copy(
                    table_v.at[idx_v.at[2 * g + i]],
                    bufs[2 * pair + i], gsems[2 * pair + i])
                for i in range(2)
            ]

        def finish_pair(g, pair, handles):
            for i in range(2):
                handles[i].wait()
                pltpu.async_copy(
                    bufs[2 * pair + i], out_slice(2 * g + i),
                    wsems[2 * pair + i])

        def drain_pair(g, pair):
            for i in range(2):
                pltpu.make_async_copy(
                    bufs[2 * pair + i], out_slice(2 * g + i),
                    wsems[2 * pair + i]).wait()

        finish_pair(0, 0, fire_pair(0, 0))
        finish_pair(1, 1, fire_pair(1, 1))

        def body(k, carry):
            g0 = 2 * k
            drain_pair(g0 - 2, 0)
            h0 = fire_pair(g0, 0)
            drain_pair(g0 - 1, 1)
            h1 = fire_pair(g0 + 1, 1)
            finish_pair(g0, 0, h0)
            finish_pair(g0 + 1, 1, h1)
            return carry

        n_pairs = chunks_per_w // 2
        lax.fori_loop(1, n_pairs // 2, body, 0)

        drain_pair(n_pairs - 2, 0)
        drain_pair(n_pairs - 1, 1)

    return gather_kernel


def kernel(x, table):
    orig_shape = x.shape
    n_rows = x.size
    idx = x.reshape(n_rows // CHUNK, CHUNK).astype(jnp.int32)
    out = _make_kernel(n_rows)(table, idx)
    return out.reshape(*orig_shape, EMBEDDING_LENGTH)
